# baseline (speedup 1.0000x reference)
import functools

import jax
import jax.numpy as jnp
from jax.experimental import pallas as pl
from jax.experimental.pallas import tpu as pltpu

_EPS = 1e-08
_NEG = -3.0e38


def _body(K, mem_ref, keys_ref, beta_ref, wt_ref, bias_ref, out_ref, w_ref,
          cand_ref):
    keys = keys_ref[0]
    beta = beta_ref[0]
    H, D = keys.shape
    R = mem_ref.shape[1]
    N = R * 2
    NB = N // 128
    HB = NB // 2
    mem2 = mem_ref[0]

    k_sq = jnp.sum(keys * keys, axis=-1, keepdims=True)
    k_norm = jnp.maximum(jnp.sqrt(k_sq), _EPS)

    CH = 8
    rows = R // CH
    msqE_parts, msqO_parts = [], []
    for c in range(CH):
        ch = mem2[c * rows:(c + 1) * rows, :]
        sq = (ch * ch).reshape(rows // 128, 128, 128)
        msqE_parts.append(jnp.sum(sq[:, :, :D], axis=-1))
        msqO_parts.append(jnp.sum(sq[:, :, D:], axis=-1))
    m_sq = jnp.concatenate(msqE_parts + msqO_parts, axis=0)
    m_norm = jnp.maximum(jnp.sqrt(m_sq), _EPS)

    zeros_k = jnp.zeros_like(keys)
    keysE = jnp.concatenate([keys, zeros_k], axis=1)
    keysO = jnp.concatenate([zeros_k, keys], axis=1)
    dotE = jax.lax.dot_general(keysE, mem2, (((1,), (1,)), ((), ())),
                               preferred_element_type=jnp.float32)
    dotO = jax.lax.dot_general(keysO, mem2, (((1,), (1,)), ((), ())),
                               preferred_element_type=jnp.float32)
    dot3 = jnp.concatenate([dotE.reshape(H, HB, 128),
                            dotO.reshape(H, HB, 128)], axis=1)
    denom3 = k_norm[:, :, None] * m_norm[None, :, :] + _EPS
    sim3 = dot3 / denom3 * beta.reshape(H, 1, 1)

    bm0 = jnp.max(sim3, axis=-1)
    row_max = jnp.max(bm0, axis=-1, keepdims=True)

    rank_gt = jnp.zeros((H, NB), jnp.float32)
    for c in range(4):
        seg = bm0[:, c * (NB // 4):(c + 1) * (NB // 4)]
        rank_gt = rank_gt + jnp.sum(
            (seg[:, None, :] > bm0[:, :, None]).astype(jnp.float32), axis=-1)
    bt = jnp.min(jnp.where(rank_gt < K, bm0, 3.4e38), axis=-1,
                 keepdims=True)
    lt_incl_nb = (jax.lax.broadcasted_iota(jnp.int32, (NB, NB), 0)
                  <= jax.lax.broadcasted_iota(jnp.int32, (NB, NB), 1)
                  ).astype(jnp.float32)
    gtb = bm0 > bt
    eqb = (bm0 == bt).astype(jnp.float32)
    n_gtb = jnp.sum(gtb.astype(jnp.float32), axis=-1, keepdims=True)
    rank_eq = jax.lax.dot_general(eqb, lt_incl_nb, (((1,), (0,)), ((), ())),
                                  preferred_element_type=jnp.float32)
    mask_sel = jnp.where(gtb | ((eqb > 0.0) & (rank_eq <= K - n_gtb)),
                         1.0, 0.0)
    slot = jax.lax.dot_general(mask_sel, lt_incl_nb, (((1,), (0,)), ((), ())),
                               preferred_element_type=jnp.float32)
    jio = jax.lax.broadcasted_iota(jnp.int32, (H, K, NB), 1).astype(jnp.float32)
    onehots = jnp.where((slot[:, None, :] == jio + 1.0)
                        & (mask_sel[:, None, :] > 0.0), 1.0, 0.0)

    for h in range(H):
        cand_ref[h] = jax.lax.dot_general(
            onehots[h], sim3[h], (((1,), (0,)), ((), ())),
            precision=jax.lax.Precision.HIGHEST,
            preferred_element_type=jnp.float32)
    cand = cand_ref[...].reshape(H, K * 128)

    def step(_, carry):
        s, thr, found = carry
        cur = jnp.max(s, axis=-1, keepdims=True)
        hit = s >= cur
        c = jnp.sum(hit.astype(jnp.float32), axis=-1, keepdims=True)
        thr = jnp.where(found < K, cur, thr)
        found = found + c
        s = jnp.where(hit, _NEG, s)
        return s, thr, found

    _, thr2, _ = jax.lax.fori_loop(
        0, K, step, (cand, row_max, jnp.zeros((H, 1), jnp.float32)))
    thr = thr2[:, :, None]
    rm2 = row_max

    cgt = (cand > thr2).astype(jnp.float32)
    n_gt2 = jnp.sum(cgt, axis=-1, keepdims=True)
    m = (K - n_gt2)[:, :, None]
    zsum = jnp.sum(cgt * jnp.exp(cand - rm2), axis=-1, keepdims=True)
    z2 = zsum + (K - n_gt2) * jnp.exp(thr2 - rm2)

    cand3 = cand_ref[...]
    ceq = cand3 == thr
    iobf = jio[:, 0, :]
    bsel = jnp.sum(onehots * iobf[:, None, :], axis=-1)
    plane = jnp.where(bsel >= HB, 1.0, 0.0)
    creal = bsel - plane * HB
    lane = jax.lax.broadcasted_iota(jnp.int32, (H, K, 128), 2).astype(
        jnp.float32)
    oidx = 2.0 * (128.0 * creal[:, :, None] + lane) + plane[:, :, None]

    def tie_step(j, chosen):
        open_ = ceq & (chosen == 0.0)
        candidx = jnp.where(open_, oidx, 3.4e38)
        mn = jnp.min(jnp.min(candidx, axis=-1), axis=-1)[:, None, None]
        take = (oidx == mn) & open_ & (jnp.float32(j) < m)
        return chosen + jnp.where(take, 1.0, 0.0)

    tiesel = jax.lax.fori_loop(0, K, tie_step,
                               jnp.zeros((H, K, 128), jnp.float32))

    for h in range(H):
        tie_h = jax.lax.dot_general(onehots[h], tiesel[h],
                                    (((0,), (0,)), ((), ())),
                                    preferred_element_type=jnp.float32)
        sim_h = sim3[h]
        rm_h = row_max[h:h + 1]
        thr_h = thr2[h:h + 1]
        z_h = z2[h:h + 1]
        e_h = jnp.where(sim_h > thr_h, jnp.exp(sim_h - rm_h), 0.0)
        w_ref[0, h] = (e_h + tie_h * jnp.exp(thr_h - rm_h)) / z_h

    wts = w_ref[0].reshape(H, N)
    wtsE = wts[:, :R]
    wtsO = wts[:, R:]
    readE = jax.lax.dot_general(wtsE, mem2, (((1,), (0,)), ((), ())),
                                preferred_element_type=jnp.float32)
    readO = jax.lax.dot_general(wtsO, mem2, (((1,), (0,)), ((), ())),
                                preferred_element_type=jnp.float32)
    read = readE[:, :D] + readO[:, D:]
    per_head = jax.lax.dot_general(read, wt_ref[...],
                                   (((1,), (2,)), ((0,), (0,))),
                                   preferred_element_type=jnp.float32)
    out_ref[0] = jnp.sum(per_head, axis=0, keepdims=True) + bias_ref[...]


@jax.jit
def kernel(memory, read_keys, beta, W, b):
    B, N, D = memory.shape
    H = read_keys.shape[1]
    K = 32
    NB = N // 128

    beta3 = beta.reshape(B, 1, H)
    Wt = W.reshape(D, H, D).transpose(1, 0, 2)
    b2 = b.reshape(1, D)

    grid = (B,)
    out_shapes = (
        jax.ShapeDtypeStruct((B, 1, D), jnp.float32),
        jax.ShapeDtypeStruct((B, H, NB, 128), jnp.float32),
    )
    read_combined, weights_pos = pl.pallas_call(
        functools.partial(_body, K),
        grid=grid,
        in_specs=[
            pl.BlockSpec((1, N * D // 128, 128), lambda i: (i, 0, 0)),
            pl.BlockSpec((1, H, D), lambda i: (i, 0, 0)),
            pl.BlockSpec((1, 1, H), lambda i: (i, 0, 0)),
            pl.BlockSpec((H, D, D), lambda i: (0, 0, 0)),
            pl.BlockSpec((1, D), lambda i: (0, 0)),
        ],
        out_specs=(
            pl.BlockSpec((1, 1, D), lambda i: (i, 0, 0)),
            pl.BlockSpec((1, H, NB, 128), lambda i: (i, 0, 0, 0)),
        ),
        out_shape=out_shapes,
        scratch_shapes=[
            pltpu.VMEM((H, K, 128), jnp.float32),
        ],
        compiler_params=pltpu.CompilerParams(
            vmem_limit_bytes=60 * 1024 * 1024),
    )(memory.reshape(B, N * D // 128, 128), read_keys, beta3, Wt, b2)
    weights = (weights_pos.reshape(B, H, 2, N // 2)
               .transpose(0, 1, 3, 2).reshape(B, H, N))
    return (read_combined.reshape(B, D), weights)


# speedup vs baseline: 2.7958x; 2.7958x over previous
import functools

import jax
import jax.numpy as jnp
from jax.experimental import pallas as pl
from jax.experimental.pallas import tpu as pltpu

_EPS = 1e-08
_NEG = -3.0e38


def _body(K, mem_ref, keys_ref, beta_ref, wt_ref, bias_ref, out_ref, w_ref,
          osc_ref, cand_ref, msq_ref):
    mem = mem_ref[0]
    keys = keys_ref[0]
    beta = beta_ref[0]
    N, D = mem.shape
    H = keys.shape[0]
    NB = N // 128

    k_sq = jnp.sum(keys * keys, axis=-1, keepdims=True)
    k_norm = jnp.maximum(jnp.sqrt(k_sq), _EPS)

    CH = 8
    rows = N // CH
    for c in range(CH):
        chunk = mem_ref[0, pl.ds(c * rows, rows), :]
        ch3 = chunk.reshape(rows // 128, 128, D)
        msq_ref[pl.ds(c * (rows // 128), rows // 128), :] = jnp.sum(
            ch3 * ch3, axis=-1)
    m_norm = jnp.maximum(jnp.sqrt(msq_ref[...]), _EPS)

    dot = jax.lax.dot_general(keys, mem, (((1,), (1,)), ((), ())),
                              preferred_element_type=jnp.float32)
    dot3 = dot.reshape(H, NB, 128)
    denom3 = k_norm[:, :, None] * m_norm[None, :, :] + _EPS
    sim3 = dot3 / denom3 * beta.reshape(H, 1, 1)

    bm0 = jnp.max(sim3, axis=-1)
    row_max = jnp.max(bm0, axis=-1, keepdims=True)

    iob = jax.lax.broadcasted_iota(jnp.int32, (H, NB), 1)

    def blk_step(j, bm):
        cur = jnp.max(bm, axis=-1, keepdims=True)
        candi = jnp.where(bm >= cur, iob, NB)
        csel = jnp.min(candi, axis=-1, keepdims=True)
        onehot = (iob == csel).astype(jnp.float32)
        osc_ref[:, pl.ds(j, 1), :] = onehot[:, None, :]
        bm = jnp.where(onehot > 0.0, _NEG, bm)
        return bm

    jax.lax.fori_loop(0, K, blk_step, bm0)

    for h in range(H):
        cand_ref[h] = jax.lax.dot_general(
            osc_ref[h], sim3[h], (((1,), (0,)), ((), ())),
            precision=jax.lax.Precision.HIGHEST,
            preferred_element_type=jnp.float32)
    cand = cand_ref[...]

    def step(_, carry):
        s, thr, found = carry
        cur = jnp.max(jnp.max(s, axis=-1), axis=-1)[:, None, None]
        hit = s >= cur
        c = jnp.sum(jnp.sum(hit.astype(jnp.float32), axis=-1),
                    axis=-1)[:, None, None]
        thr = jnp.where(found < K, cur, thr)
        found = found + c
        s = jnp.where(hit, _NEG, s)
        return s, thr, found

    _, thr, _ = jax.lax.fori_loop(
        0, K, step, (cand, row_max[:, :, None],
                     jnp.zeros((H, 1, 1), jnp.float32)))

    rm3 = row_max[:, :, None]
    cgt = (cand > thr).astype(jnp.float32)
    n_gt = jnp.sum(jnp.sum(cgt, axis=-1), axis=-1)[:, None, None]
    m = K - n_gt
    zsum = jnp.sum(jnp.sum(cgt * jnp.exp(cand - rm3), axis=-1),
                   axis=-1)[:, None, None]
    z = zsum + m * jnp.exp(thr - rm3)

    eq3 = (sim3 == thr).astype(jnp.float32)
    lt_incl = (jax.lax.broadcasted_iota(jnp.int32, (128, 128), 0)
               <= jax.lax.broadcasted_iota(jnp.int32, (128, 128), 1)
               ).astype(jnp.float32)
    intra = jax.lax.dot_general(eq3.reshape(H * NB, 128), lt_incl,
                                (((1,), (0,)), ((), ())),
                                preferred_element_type=jnp.float32)
    intra3 = intra.reshape(H, NB, 128)
    bs = jnp.sum(eq3, axis=-1)
    lt_incl_nb = (jax.lax.broadcasted_iota(jnp.int32, (NB, NB), 0)
                  <= jax.lax.broadcasted_iota(jnp.int32, (NB, NB), 1)
                  ).astype(jnp.float32)
    bpre = jax.lax.dot_general(bs, lt_incl_nb, (((1,), (0,)), ((), ())),
                               preferred_element_type=jnp.float32) - bs
    pre3 = intra3 + bpre[:, :, None]
    selected = (sim3 > thr) | ((eq3 > 0.0) & (pre3 <= m))
    wts3 = jnp.where(selected, jnp.exp(sim3 - rm3), 0.0) / z
    w_ref[0] = wts3

    wts = wts3.reshape(H, N)
    read = jax.lax.dot_general(wts, mem, (((1,), (0,)), ((), ())),
                               preferred_element_type=jnp.float32)
    per_head = jax.lax.dot_general(read, wt_ref[...],
                                   (((1,), (2,)), ((0,), (0,))),
                                   preferred_element_type=jnp.float32)
    out_ref[0] = jnp.sum(per_head, axis=0, keepdims=True) + bias_ref[...]


@jax.jit
def kernel(memory, read_keys, beta, W, b):
    B, N, D = memory.shape
    H = read_keys.shape[1]
    K = 32
    NB = N // 128

    beta3 = beta.reshape(B, 1, H)
    Wt = W.reshape(D, H, D).transpose(1, 0, 2)
    b2 = b.reshape(1, D)

    grid = (B,)
    out_shapes = (
        jax.ShapeDtypeStruct((B, 1, D), jnp.float32),
        jax.ShapeDtypeStruct((B, H, NB, 128), jnp.float32),
    )
    read_combined, weights = pl.pallas_call(
        functools.partial(_body, K),
        grid=grid,
        in_specs=[
            pl.BlockSpec((1, N, D), lambda i: (i, 0, 0)),
            pl.BlockSpec((1, H, D), lambda i: (i, 0, 0)),
            pl.BlockSpec((1, 1, H), lambda i: (i, 0, 0)),
            pl.BlockSpec((H, D, D), lambda i: (0, 0, 0)),
            pl.BlockSpec((1, D), lambda i: (0, 0)),
        ],
        out_specs=(
            pl.BlockSpec((1, 1, D), lambda i: (i, 0, 0)),
            pl.BlockSpec((1, H, NB, 128), lambda i: (i, 0, 0, 0)),
        ),
        out_shape=out_shapes,
        scratch_shapes=[
            pltpu.VMEM((H, K, NB), jnp.float32),
            pltpu.VMEM((H, K, 128), jnp.float32),
            pltpu.VMEM((NB, 128), jnp.float32),
        ],
        compiler_params=pltpu.CompilerParams(
            vmem_limit_bytes=60 * 1024 * 1024),
    )(memory, read_keys, beta3, Wt, b2)
    return (read_combined.reshape(B, D), weights.reshape(B, H, N))
